# Initial kernel scaffold; baseline (speedup 1.0000x reference)
import jax
import jax.numpy as jnp
from jax.experimental import pallas as pl


def kernel(movie_ids, genre_tokens, movie_table, genre_table):
    raise NotImplementedError("write your pallas kernel here")


# speedup vs baseline: 15.9952x; 15.9952x over previous
import jax
import jax.numpy as jnp
from jax import lax
from jax.experimental import pallas as pl
from jax.experimental.pallas import tpu as pltpu
from jax.experimental.pallas import tpu_sc as plsc

_D = 32
_B = 16384
_L = 20

_NW = 32
_ROWS_W = _B // _NW
_C = 64
_NCHUNK = _ROWS_W // _C
_STEP = 128
_SPC = (_C * _L) // _STEP
_MSTEP = _ROWS_W // _STEP


def _sc_body(mi_hbm, tok_hbm, mt_hbm, gt_hbm, out_hbm,
             mi_v, tok_v, t0_v, mrows, g0, g1, cb0, cb1, nzf_v, inv_v,
             sem_m, sem_g0, sem_g1, sem_o0, sem_o1):
    wid = lax.axis_index("s") * 2 + lax.axis_index("c")
    base = wid * _ROWS_W

    pltpu.sync_copy(mi_hbm.at[wid], mi_v)
    pltpu.sync_copy(tok_hbm.at[wid], tok_v)
    pltpu.sync_copy(gt_hbm.at[0], t0_v)

    mdesc = [pltpu.async_copy(mt_hbm.at[mi_v.at[s]],
                              mrows.at[pl.ds(s * _STEP, _STEP)], sem_m)
             for s in range(_MSTEP)]

    gbufs, gsems = (g0, g1), (sem_g0, sem_g1)
    combs, osems = (cb0, cb1), (sem_o0, sem_o1)

    def fire(kk):
        buf = gbufs[kk % 2]
        return [pltpu.async_copy(gt_hbm.at[tok_v.at[kk * _SPC + s]],
                                 buf.at[pl.ds(s * _STEP, _STEP)],
                                 gsems[kk % 2])
                for s in range(_SPC)]

    pending = {0: fire(0)}
    odesc = [None, None]

    t0a = t0_v[pl.ds(0, 16)]
    t0b = t0_v[pl.ds(16, 16)]

    for kk in range(_NCHUNK):
        if kk + 1 < _NCHUNK:
            pending[kk + 1] = fire(kk + 1)
        for dsc in pending.pop(kk):
            dsc.wait()
        if kk % 2 == 0:
            mdesc[kk // 2].wait()
        buf = gbufs[kk % 2]
        comb = combs[kk % 2]
        if odesc[kk % 2] is not None:
            odesc[kk % 2].wait()

        for g in range(_C // 16):
            nzf = jnp.zeros((16,), jnp.float32)
            for l in range(_L):
                pos = l * _C + g * 16
                t = tok_v[kk * _SPC + pos // _STEP, pl.ds(pos % _STEP, 16)]
                nzf = nzf + jnp.where(t == 0, 1.0, 0.0)
            nzf_v[pl.ds(g * 16, 16)] = nzf
            inv_v[pl.ds(g * 16, 16)] = 1.0 / jnp.maximum(_L - nzf, 1.0)

        def row_body(c, carry):
            acc0 = buf[c, pl.ds(0, 16)]
            acc1 = buf[c, pl.ds(16, 16)]
            for l in range(1, _L):
                acc0 = acc0 + buf[l * _C + c, pl.ds(0, 16)]
                acc1 = acc1 + buf[l * _C + c, pl.ds(16, 16)]
            nzc = nzf_v[pl.ds(c, 16)][0]
            ivc = inv_v[pl.ds(c, 16)][0]
            comb[c, pl.ds(0, 16)] = mrows[kk * _C + c, pl.ds(0, 16)]
            comb[c, pl.ds(16, 16)] = mrows[kk * _C + c, pl.ds(16, 16)]
            comb[c, pl.ds(_D, 16)] = (acc0 - nzc * t0a) * ivc
            comb[c, pl.ds(_D + 16, 16)] = (acc1 - nzc * t0b) * ivc
            return carry
        lax.fori_loop(0, _C, row_body, 0)

        odesc[kk % 2] = pltpu.async_copy(
            comb, out_hbm.at[pl.ds(base + kk * _C, _C)], osems[kk % 2])

    for dsc in odesc:
        if dsc is not None:
            dsc.wait()


def kernel(movie_ids, genre_tokens, movie_table, genre_table):
    mi = movie_ids.astype(jnp.int32).reshape(_NW, _MSTEP, _STEP)
    tok = genre_tokens.astype(jnp.int32).reshape(_NW, _NCHUNK, _C, _L)
    tok = tok.transpose(0, 1, 3, 2).reshape(_NW, _NCHUNK * _SPC, _STEP)
    mesh = plsc.VectorSubcoreMesh(core_axis_name="c", subcore_axis_name="s")
    run = pl.kernel(
        _sc_body,
        mesh=mesh,
        compiler_params=pltpu.CompilerParams(use_tc_tiling_on_sc=False),
        out_type=jax.ShapeDtypeStruct((_B, 2 * _D), jnp.float32),
        scratch_types=[
            pltpu.VMEM((_MSTEP, _STEP), jnp.int32),
            pltpu.VMEM((_NCHUNK * _SPC, _STEP), jnp.int32),
            pltpu.VMEM((_D,), jnp.float32),
            pltpu.VMEM((_ROWS_W, _D), jnp.float32),
            pltpu.VMEM((_C * _L, _D), jnp.float32),
            pltpu.VMEM((_C * _L, _D), jnp.float32),
            pltpu.VMEM((_C, 2 * _D), jnp.float32),
            pltpu.VMEM((_C, 2 * _D), jnp.float32),
            pltpu.VMEM((_C + 16,), jnp.float32),
            pltpu.VMEM((_C + 16,), jnp.float32),
            pltpu.SemaphoreType.DMA,
            pltpu.SemaphoreType.DMA,
            pltpu.SemaphoreType.DMA,
            pltpu.SemaphoreType.DMA,
            pltpu.SemaphoreType.DMA,
        ],
    )
    return run(mi, tok, movie_table, genre_table)
